# baseline (speedup 1.0000x reference)
import functools

import jax
import jax.numpy as jnp
from jax import lax
from jax.experimental import pallas as pl
from jax.experimental.pallas import tpu as pltpu
from jax.experimental.pallas import tpu_sc as plsc

NC = 2
NS = 16
NW = NC * NS
LANES = 16


def _make_spmm(n, d, e, dtype):
    epw = e // NW
    chunk = 80
    nchunk = epw // chunk
    rps = (n // NS) // 8 * 8
    rem = n - rps * NS
    mesh = plsc.VectorSubcoreMesh(core_axis_name="c", subcore_axis_name="s")

    BUF = 3
    H = next(h for h in range(BUF - 1, 3 * BUF)
             if (nchunk - 3 - h) % BUF == 0 and nchunk - 3 - h >= 0)
    G = (nchunk - 3 - H) // BUF

    dp = d // 2

    @functools.partial(
        pl.kernel,
        mesh=mesh,
        out_type=jax.ShapeDtypeStruct((NC, n, d), dtype),
        compiler_params=pltpu.CompilerParams(use_tc_tiling_on_sc=False,
                                             needs_layout_passes=False),
        scratch_types=[
            pltpu.VMEM_SHARED((n, d), dtype),
            pltpu.VMEM((BUF, chunk), jnp.int32),
            pltpu.VMEM((BUF, chunk), jnp.int32),
            pltpu.VMEM((BUF, chunk), dtype),
            pltpu.VMEM((chunk, dp), jnp.int32),
            pltpu.VMEM((chunk, dp), jnp.int32),
            pltpu.VMEM((chunk, dp), jnp.int32),
            pltpu.VMEM((chunk, d), dtype),
            pltpu.VMEM((chunk, d), dtype),
            pltpu.VMEM((chunk, d), dtype),
            pltpu.SemaphoreType.DMA((BUF,)),
            pltpu.SemaphoreType.DMA((BUF,)),
            pltpu.SemaphoreType.DMA((BUF,)),
            pltpu.SemaphoreType.DMA((BUF,)),
        ],
    )
    def spmm(h_hbm, src_hbm, dst_hbm, lv_hbm, z_hbm, out_hbm,
             acc, src_v, dst_v, lv_v, rows0, rows1, rows2,
             rf0, rf1, rf2, isem, jsem, gsem, ssem):
        cid = lax.axis_index("c")
        sid = lax.axis_index("s")
        wid = sid * NC + cid
        ebase = wid * epw
        rows = (rows0, rows1, rows2)
        rf = (rf0, rf1, rf2)

        pltpu.sync_copy(z_hbm.at[pl.ds(0, rps)], acc.at[pl.ds(sid * rps, rps)])
        if rem:
            @pl.when(sid == NS - 1)
            def _zero_tail():
                pltpu.sync_copy(z_hbm.at[pl.ds(0, rem)],
                                acc.at[pl.ds(rps * NS, rem)])
        plsc.subcore_barrier()

        def scale(rows_b, rf_b, lv_ref):
            def grp(g, c2):
                w16 = lv_ref[pl.ds(g * LANES, LANES)]
                for jj in range(LANES):
                    wj = w16[jj]
                    row = g * LANES + jj
                    for q in range(d // (2 * LANES)):
                        v = rows_b[row, pl.ds(q * LANES, LANES)]
                        a = plsc.bitcast(v << 16, jnp.float32)
                        b2 = plsc.bitcast(v & jnp.int32(-65536), jnp.float32)
                        rf_b[row, pl.ds(2 * q * LANES, LANES)] = a * wj
                        rf_b[row, pl.ds((2 * q + 1) * LANES, LANES)] = b2 * wj
                return c2
            lax.fori_loop(0, chunk // LANES, grp, 0)

        def gather_issue(i, b):
            pltpu.async_copy(h_hbm.at[src_v.at[b]], rows[b], gsem.at[b])

        def scatter_issue(b):
            pltpu.async_copy(rf[b], acc.at[dst_v.at[b]], ssem.at[b],
                             add=True)

        def wait_scatter(b):
            pltpu.make_async_copy(rf[b], acc.at[dst_v.at[b]],
                                  ssem.at[b]).wait()

        def wait_gather(b):
            pltpu.make_async_copy(h_hbm.at[src_v.at[b]], rows[b],
                                  gsem.at[b]).wait()

        def issue_src(i, b):
            off = ebase + i * chunk
            pltpu.async_copy(src_hbm.at[pl.ds(off, chunk)],
                             src_v.at[b], isem.at[b])

        def wait_src(i, b):
            off = ebase + i * chunk
            pltpu.make_async_copy(src_hbm.at[pl.ds(off, chunk)],
                                  src_v.at[b], isem.at[b]).wait()

        def issue_dl(i, b):
            off = ebase + i * chunk
            pltpu.async_copy(dst_hbm.at[pl.ds(off, chunk)],
                             dst_v.at[b], jsem.at[b])
            pltpu.async_copy(lv_hbm.at[pl.ds(off, chunk)],
                             lv_v.at[b], jsem.at[b])

        def wait_dstlv(i, b):
            off = ebase + i * chunk
            pltpu.make_async_copy(dst_hbm.at[pl.ds(off, chunk)],
                                  dst_v.at[b], jsem.at[b]).wait()
            pltpu.make_async_copy(lv_hbm.at[pl.ds(off, chunk)],
                                  lv_v.at[b], jsem.at[b]).wait()

        pltpu.sync_copy(src_hbm.at[pl.ds(ebase, chunk)], src_v.at[0])
        gather_issue(0, 0)
        pltpu.sync_copy(src_hbm.at[pl.ds(ebase + chunk, chunk)],
                        src_v.at[1])
        gather_issue(1, 1)
        issue_src(2, 2)
        issue_dl(0, 0)
        issue_dl(1, 1)

        def body_steps(i, b, wait_prev=True, pf_dl=True, pf_g=True,
                       pf_src=True):
            bp = (b + BUF - 1) % BUF
            if pf_g:
                wait_src(i + 2, bp)
                gather_issue(i + 2, bp)
            if wait_prev:
                wait_scatter(bp)
            if pf_dl:
                issue_dl(i + 2, bp)
            wait_gather(b)
            if pf_src:
                issue_src(i + 3, b)
            wait_dstlv(i, b)
            scale(rows[b], rf[b], lv_v.at[b])
            scatter_issue(b)

        for i in range(H):
            body_steps(i, i % BUF, wait_prev=(i >= 1))

        def group(g, carry):
            i0 = H + BUF * g
            for r in range(BUF):
                body_steps(i0 + r, (H + r) % BUF)
            return carry

        lax.fori_loop(0, G, group, 0)

        t0 = nchunk - 3
        body_steps(t0, t0 % BUF, pf_src=False)
        body_steps(t0 + 1, (t0 + 1) % BUF, pf_dl=False, pf_g=False,
                   pf_src=False)
        body_steps(t0 + 2, (t0 + 2) % BUF, pf_dl=False, pf_g=False,
                   pf_src=False)
        wait_scatter((nchunk - 1) % BUF)
        plsc.subcore_barrier()
        pltpu.sync_copy(acc.at[pl.ds(sid * rps, rps)],
                        out_hbm.at[cid, pl.ds(sid * rps, rps)])
        if rem:
            @pl.when(sid == NS - 1)
            def _out_tail():
                pltpu.sync_copy(acc.at[pl.ds(rps * NS, rem)],
                                out_hbm.at[cid, pl.ds(rps * NS, rem)])

    return spmm


def _dense_layer(h, p, w_cat, b):
    n, d = h.shape
    rblk = 1000
    grid = (n // rblk,)

    def body(h_ref, p_ref, w_ref, b_ref, hn_ref, nrm_ref):
        lh = p_ref[0] + p_ref[1]
        hv = h_ref[...]
        cat = jnp.concatenate([lh + hv, hv * lh], axis=1)
        y = jnp.dot(cat, w_ref[...], preferred_element_type=jnp.float32)
        y = y + b_ref[...]
        y = jnp.where(y >= 0, y, 0.2 * y)
        hn_ref[...] = y
        ss = jnp.sum(y * y, axis=1, keepdims=True)
        nrm_ref[...] = y * lax.rsqrt(jnp.maximum(ss, 1e-12))

    row_spec = pl.BlockSpec((rblk, d), lambda i: (i, 0))
    return pl.pallas_call(
        body,
        grid=grid,
        in_specs=[
            row_spec,
            pl.BlockSpec((2, rblk, d), lambda i: (0, i, 0)),
            pl.BlockSpec((2 * d, d), lambda i: (0, 0)),
            pl.BlockSpec((1, d), lambda i: (0, 0)),
        ],
        out_specs=[row_spec, row_spec],
        out_shape=[
            jax.ShapeDtypeStruct((n, d), h.dtype),
            jax.ShapeDtypeStruct((n, d), h.dtype),
        ],
    )(h, p, w_cat, b)


def _dense_layer_final(h, p, w_cat, b, prevs):
    n, d = h.shape
    rblk = 1000
    grid = (n // rblk,)
    nprev = len(prevs)
    dout = (nprev + 1) * d

    def body(h_ref, p_ref, w_ref, b_ref, *rest):
        prev_refs, out_ref = rest[:nprev], rest[nprev]
        lh = p_ref[0] + p_ref[1]
        hv = h_ref[...]
        cat = jnp.concatenate([lh + hv, hv * lh], axis=1)
        y = jnp.dot(cat, w_ref[...], preferred_element_type=jnp.float32)
        y = y + b_ref[...]
        y = jnp.where(y >= 0, y, 0.2 * y)
        ss = jnp.sum(y * y, axis=1, keepdims=True)
        nrm = y * lax.rsqrt(jnp.maximum(ss, 1e-12))
        out_ref[...] = jnp.concatenate(
            [r[...] for r in prev_refs] + [nrm], axis=1)

    row_spec = pl.BlockSpec((rblk, d), lambda i: (i, 0))
    return pl.pallas_call(
        body,
        grid=grid,
        in_specs=[
            row_spec,
            pl.BlockSpec((2, rblk, d), lambda i: (0, i, 0)),
            pl.BlockSpec((2 * d, d), lambda i: (0, 0)),
            pl.BlockSpec((1, d), lambda i: (0, 0)),
        ] + [row_spec] * nprev,
        out_specs=pl.BlockSpec((rblk, dout), lambda i: (i, 0)),
        out_shape=jax.ShapeDtypeStruct((n, dout), h.dtype),
    )(h, p, w_cat, b, *prevs)


def kernel(x, edge_index, L_vals, W_gc, b_gc, W_bi, b_bi):
    n, d = x.shape
    e = L_vals.shape[0]
    k = W_gc.shape[0]
    src = edge_index[0]
    dst = edge_index[1]
    zeros = jnp.zeros(((n // NS) // 8 * 8, d), x.dtype)
    spmm = _make_spmm(n, d, e, x.dtype)

    def pack_rows(hh):
        hb = hh.astype(jnp.bfloat16).reshape(n, d // 32, 2, LANES)
        arr = hb.transpose(0, 1, 3, 2)
        return lax.bitcast_convert_type(arr, jnp.int32).reshape(n, d // 2)

    h = x
    nrms = []
    for i in range(k):
        p = spmm(pack_rows(h), src, dst, L_vals, zeros)
        w_cat = jnp.concatenate([W_gc[i], W_bi[i]], axis=0)
        b = (b_gc[i] + b_bi[i]).reshape(1, d)
        if i < k - 1:
            h, nrm = _dense_layer(h, p, w_cat, b)
            nrms.append(nrm)
        else:
            out = _dense_layer_final(h, p, w_cat, b, [x] + nrms)
    return out


# speedup vs baseline: 1.0008x; 1.0008x over previous
import functools

import jax
import jax.numpy as jnp
from jax import lax
from jax.experimental import pallas as pl
from jax.experimental.pallas import tpu as pltpu
from jax.experimental.pallas import tpu_sc as plsc

NC = 2
NS = 16
NW = NC * NS
LANES = 16


def _make_spmm(n, d, e, dtype):
    epw = e // NW
    chunk = 80
    nchunk = epw // chunk
    rps = (n // NS) // 8 * 8
    rem = n - rps * NS
    mesh = plsc.VectorSubcoreMesh(core_axis_name="c", subcore_axis_name="s")

    BUF = 3
    H = next(h for h in range(BUF - 1, 3 * BUF)
             if (nchunk - 3 - h) % BUF == 0 and nchunk - 3 - h >= 0)
    G = (nchunk - 3 - H) // BUF

    dp = d // 2

    @functools.partial(
        pl.kernel,
        mesh=mesh,
        out_type=jax.ShapeDtypeStruct((NC, n, d), dtype),
        compiler_params=pltpu.CompilerParams(use_tc_tiling_on_sc=False),
        scratch_types=[
            pltpu.VMEM_SHARED((n, d), dtype),
            pltpu.VMEM((BUF, chunk), jnp.int32),
            pltpu.VMEM((BUF, chunk), jnp.int32),
            pltpu.VMEM((BUF, chunk), dtype),
            pltpu.VMEM((chunk, dp), jnp.int32),
            pltpu.VMEM((chunk, dp), jnp.int32),
            pltpu.VMEM((chunk, dp), jnp.int32),
            pltpu.VMEM((chunk, d), dtype),
            pltpu.VMEM((chunk, d), dtype),
            pltpu.VMEM((chunk, d), dtype),
            pltpu.SemaphoreType.DMA((BUF,)),
            pltpu.SemaphoreType.DMA((BUF,)),
            pltpu.SemaphoreType.DMA((BUF,)),
            pltpu.SemaphoreType.DMA((BUF,)),
        ],
    )
    def spmm(h_hbm, src_hbm, dst_hbm, lv_hbm, z_hbm, out_hbm,
             acc, src_v, dst_v, lv_v, rows0, rows1, rows2,
             rf0, rf1, rf2, isem, jsem, gsem, ssem):
        cid = lax.axis_index("c")
        sid = lax.axis_index("s")
        wid = sid * NC + cid
        ebase = wid * epw
        rows = (rows0, rows1, rows2)
        rf = (rf0, rf1, rf2)

        pltpu.sync_copy(z_hbm.at[pl.ds(0, rps)], acc.at[pl.ds(sid * rps, rps)])
        if rem:
            @pl.when(sid == NS - 1)
            def _zero_tail():
                pltpu.sync_copy(z_hbm.at[pl.ds(0, rem)],
                                acc.at[pl.ds(rps * NS, rem)])
        plsc.subcore_barrier()

        def scale(rows_b, rf_b, lv_ref):
            def grp(g, c2):
                w16 = lv_ref[pl.ds(g * LANES, LANES)]
                for jj in range(LANES):
                    wj = w16[jj]
                    row = g * LANES + jj
                    for q in range(d // (2 * LANES)):
                        v = rows_b[row, pl.ds(q * LANES, LANES)]
                        a = lax.bitcast_convert_type(v << 16, jnp.float32)
                        b2 = lax.bitcast_convert_type(
                            v & jnp.int32(-65536), jnp.float32)
                        rf_b[row, pl.ds(2 * q * LANES, LANES)] = a * wj
                        rf_b[row, pl.ds((2 * q + 1) * LANES, LANES)] = b2 * wj
                return c2
            lax.fori_loop(0, chunk // LANES, grp, 0)

        def gather_issue(i, b):
            pltpu.async_copy(h_hbm.at[src_v.at[b]], rows[b], gsem.at[b])

        def scatter_issue(b):
            pltpu.async_copy(rf[b], acc.at[dst_v.at[b]], ssem.at[b],
                             add=True)

        def wait_scatter(b):
            pltpu.make_async_copy(rf[b], acc.at[dst_v.at[b]],
                                  ssem.at[b]).wait()

        def wait_gather(b):
            pltpu.make_async_copy(h_hbm.at[src_v.at[b]], rows[b],
                                  gsem.at[b]).wait()

        def issue_src(i, b):
            off = ebase + i * chunk
            pltpu.async_copy(src_hbm.at[pl.ds(off, chunk)],
                             src_v.at[b], isem.at[b])

        def wait_src(i, b):
            off = ebase + i * chunk
            pltpu.make_async_copy(src_hbm.at[pl.ds(off, chunk)],
                                  src_v.at[b], isem.at[b]).wait()

        def issue_dl(i, b):
            off = ebase + i * chunk
            pltpu.async_copy(dst_hbm.at[pl.ds(off, chunk)],
                             dst_v.at[b], jsem.at[b])
            pltpu.async_copy(lv_hbm.at[pl.ds(off, chunk)],
                             lv_v.at[b], jsem.at[b])

        def wait_dstlv(i, b):
            off = ebase + i * chunk
            pltpu.make_async_copy(dst_hbm.at[pl.ds(off, chunk)],
                                  dst_v.at[b], jsem.at[b]).wait()
            pltpu.make_async_copy(lv_hbm.at[pl.ds(off, chunk)],
                                  lv_v.at[b], jsem.at[b]).wait()

        pltpu.sync_copy(src_hbm.at[pl.ds(ebase, chunk)], src_v.at[0])
        gather_issue(0, 0)
        pltpu.sync_copy(src_hbm.at[pl.ds(ebase + chunk, chunk)],
                        src_v.at[1])
        gather_issue(1, 1)
        issue_src(2, 2)
        issue_dl(0, 0)
        issue_dl(1, 1)

        def body_steps(i, b, wait_prev=True, pf_dl=True, pf_g=True,
                       pf_src=True):
            bp = (b + BUF - 1) % BUF
            if pf_g:
                wait_src(i + 2, bp)
                gather_issue(i + 2, bp)
            if wait_prev:
                wait_scatter(bp)
            if pf_dl:
                issue_dl(i + 2, bp)
            wait_gather(b)
            if pf_src:
                issue_src(i + 3, b)
            wait_dstlv(i, b)
            scale(rows[b], rf[b], lv_v.at[b])
            scatter_issue(b)

        for i in range(H):
            body_steps(i, i % BUF, wait_prev=(i >= 1))

        def group(g, carry):
            i0 = H + BUF * g
            for r in range(BUF):
                body_steps(i0 + r, (H + r) % BUF)
            return carry

        lax.fori_loop(0, G, group, 0)

        t0 = nchunk - 3
        body_steps(t0, t0 % BUF, pf_src=False)
        body_steps(t0 + 1, (t0 + 1) % BUF, pf_dl=False, pf_g=False,
                   pf_src=False)
        body_steps(t0 + 2, (t0 + 2) % BUF, pf_dl=False, pf_g=False,
                   pf_src=False)
        wait_scatter((nchunk - 1) % BUF)
        plsc.subcore_barrier()
        pltpu.sync_copy(acc.at[pl.ds(sid * rps, rps)],
                        out_hbm.at[cid, pl.ds(sid * rps, rps)])
        if rem:
            @pl.when(sid == NS - 1)
            def _out_tail():
                pltpu.sync_copy(acc.at[pl.ds(rps * NS, rem)],
                                out_hbm.at[cid, pl.ds(rps * NS, rem)])

    return spmm


def _dense_layer(h, p, w_cat, b):
    n, d = h.shape
    rblk = 1000
    grid = (n // rblk,)

    def body(h_ref, p_ref, w_ref, b_ref, hn_ref, nrm_ref):
        lh = p_ref[0] + p_ref[1]
        hv = h_ref[...]
        cat = jnp.concatenate([lh + hv, hv * lh], axis=1)
        y = jnp.dot(cat, w_ref[...], preferred_element_type=jnp.float32)
        y = y + b_ref[...]
        y = jnp.where(y >= 0, y, 0.2 * y)
        hn_ref[...] = y
        ss = jnp.sum(y * y, axis=1, keepdims=True)
        nrm_ref[...] = y * lax.rsqrt(jnp.maximum(ss, 1e-12))

    row_spec = pl.BlockSpec((rblk, d), lambda i: (i, 0))
    return pl.pallas_call(
        body,
        grid=grid,
        in_specs=[
            row_spec,
            pl.BlockSpec((2, rblk, d), lambda i: (0, i, 0)),
            pl.BlockSpec((2 * d, d), lambda i: (0, 0)),
            pl.BlockSpec((1, d), lambda i: (0, 0)),
        ],
        out_specs=[row_spec, row_spec],
        out_shape=[
            jax.ShapeDtypeStruct((n, d), h.dtype),
            jax.ShapeDtypeStruct((n, d), h.dtype),
        ],
    )(h, p, w_cat, b)


def _dense_layer_final(h, p, w_cat, b, prevs):
    n, d = h.shape
    rblk = 1000
    grid = (n // rblk,)
    nprev = len(prevs)
    dout = (nprev + 1) * d

    def body(h_ref, p_ref, w_ref, b_ref, *rest):
        prev_refs, out_ref = rest[:nprev], rest[nprev]
        lh = p_ref[0] + p_ref[1]
        hv = h_ref[...]
        cat = jnp.concatenate([lh + hv, hv * lh], axis=1)
        y = jnp.dot(cat, w_ref[...], preferred_element_type=jnp.float32)
        y = y + b_ref[...]
        y = jnp.where(y >= 0, y, 0.2 * y)
        ss = jnp.sum(y * y, axis=1, keepdims=True)
        nrm = y * lax.rsqrt(jnp.maximum(ss, 1e-12))
        out_ref[...] = jnp.concatenate(
            [r[...] for r in prev_refs] + [nrm], axis=1)

    row_spec = pl.BlockSpec((rblk, d), lambda i: (i, 0))
    return pl.pallas_call(
        body,
        grid=grid,
        in_specs=[
            row_spec,
            pl.BlockSpec((2, rblk, d), lambda i: (0, i, 0)),
            pl.BlockSpec((2 * d, d), lambda i: (0, 0)),
            pl.BlockSpec((1, d), lambda i: (0, 0)),
        ] + [row_spec] * nprev,
        out_specs=pl.BlockSpec((rblk, dout), lambda i: (i, 0)),
        out_shape=jax.ShapeDtypeStruct((n, dout), h.dtype),
    )(h, p, w_cat, b, *prevs)


def kernel(x, edge_index, L_vals, W_gc, b_gc, W_bi, b_bi):
    n, d = x.shape
    e = L_vals.shape[0]
    k = W_gc.shape[0]
    src = edge_index[0]
    dst = edge_index[1]
    zeros = jnp.zeros(((n // NS) // 8 * 8, d), x.dtype)
    spmm = _make_spmm(n, d, e, x.dtype)

    def pack_rows(hh):
        hb = hh.astype(jnp.bfloat16).reshape(n, d // 32, 2, LANES)
        arr = hb.transpose(0, 1, 3, 2)
        return lax.bitcast_convert_type(arr, jnp.int32).reshape(n, d // 2)

    h = x
    nrms = []
    for i in range(k):
        p = spmm(pack_rows(h), src, dst, L_vals, zeros)
        w_cat = jnp.concatenate([W_gc[i], W_bi[i]], axis=0)
        b = (b_gc[i] + b_bi[i]).reshape(1, d)
        if i < k - 1:
            h, nrm = _dense_layer(h, p, w_cat, b)
            nrms.append(nrm)
        else:
            out = _dense_layer_final(h, p, w_cat, b, [x] + nrms)
    return out


# speedup vs baseline: 1.2024x; 1.2014x over previous
import functools

import jax
import jax.numpy as jnp
from jax import lax
from jax.experimental import pallas as pl
from jax.experimental.pallas import tpu as pltpu
from jax.experimental.pallas import tpu_sc as plsc

NC = 2
NS = 16
NW = NC * NS
LANES = 16


def _make_spmm(n, d, e, dtype):
    chunk = 128
    epw = -(-(e // NW) // chunk) * chunk
    nchunk = epw // chunk
    rps = (n // NS) // 8 * 8
    rem = n - rps * NS
    mesh = plsc.VectorSubcoreMesh(core_axis_name="c", subcore_axis_name="s")

    BUF = 3
    H = next(h for h in range(BUF - 1, 3 * BUF)
             if (nchunk - 3 - h) % BUF == 0 and nchunk - 3 - h >= 0)
    G = (nchunk - 3 - H) // BUF

    @functools.partial(
        pl.kernel,
        mesh=mesh,
        out_type=jax.ShapeDtypeStruct((NC, n, d), dtype),
        scratch_types=[
            pltpu.VMEM_SHARED((n, d), dtype),
            pltpu.VMEM((BUF, chunk), jnp.int32),
            pltpu.VMEM((BUF, chunk), jnp.int32),
            pltpu.VMEM((BUF, chunk), dtype),
            pltpu.VMEM((chunk, d), dtype),
            pltpu.VMEM((chunk, d), dtype),
            pltpu.VMEM((chunk, d), dtype),
            pltpu.SemaphoreType.DMA((BUF,)),
            pltpu.SemaphoreType.DMA((BUF,)),
            pltpu.SemaphoreType.DMA((BUF,)),
            pltpu.SemaphoreType.DMA((BUF,)),
        ],
    )
    def spmm(h_hbm, src_hbm, dst_hbm, lv_hbm, z_hbm, out_hbm,
             acc, src_v, dst_v, lv_v, rows0, rows1, rows2,
             isem, jsem, gsem, ssem):
        cid = lax.axis_index("c")
        sid = lax.axis_index("s")
        wid = sid * NC + cid
        ebase = wid * epw
        rows = (rows0, rows1, rows2)

        pltpu.sync_copy(z_hbm.at[pl.ds(0, rps)], acc.at[pl.ds(sid * rps, rps)])
        if rem:
            @pl.when(sid == NS - 1)
            def _zero_tail():
                pltpu.sync_copy(z_hbm.at[pl.ds(0, rem)],
                                acc.at[pl.ds(rps * NS, rem)])
        plsc.subcore_barrier()

        def scale(rows_b, lv_ref):
            def grp(g, c2):
                w16 = lv_ref[pl.ds(g * LANES, LANES)]
                for jj in range(LANES):
                    wj = w16[jj]
                    row = g * LANES + jj
                    for j in range(d // LANES):
                        sl = pl.ds(j * LANES, LANES)
                        rows_b[row, sl] = rows_b[row, sl] * wj
                return c2
            lax.fori_loop(0, chunk // LANES, grp, 0)

        def gather_issue(i, b):
            pltpu.async_copy(h_hbm.at[src_v.at[b]], rows[b], gsem.at[b])

        def scatter_issue(b):
            pltpu.async_copy(rows[b], acc.at[dst_v.at[b]], ssem.at[b],
                             add=True)

        def wait_scatter(b):
            pltpu.make_async_copy(rows[b], acc.at[dst_v.at[b]],
                                  ssem.at[b]).wait()

        def wait_gather(b):
            pltpu.make_async_copy(h_hbm.at[src_v.at[b]], rows[b],
                                  gsem.at[b]).wait()

        def issue_src(i, b):
            off = ebase + i * chunk
            pltpu.async_copy(src_hbm.at[pl.ds(off, chunk)],
                             src_v.at[b], isem.at[b])

        def wait_src(i, b):
            off = ebase + i * chunk
            pltpu.make_async_copy(src_hbm.at[pl.ds(off, chunk)],
                                  src_v.at[b], isem.at[b]).wait()

        def issue_dl(i, b):
            off = ebase + i * chunk
            pltpu.async_copy(dst_hbm.at[pl.ds(off, chunk)],
                             dst_v.at[b], jsem.at[b])
            pltpu.async_copy(lv_hbm.at[pl.ds(off, chunk)],
                             lv_v.at[b], jsem.at[b])

        def wait_dstlv(i, b):
            off = ebase + i * chunk
            pltpu.make_async_copy(dst_hbm.at[pl.ds(off, chunk)],
                                  dst_v.at[b], jsem.at[b]).wait()
            pltpu.make_async_copy(lv_hbm.at[pl.ds(off, chunk)],
                                  lv_v.at[b], jsem.at[b]).wait()

        pltpu.sync_copy(src_hbm.at[pl.ds(ebase, chunk)], src_v.at[0])
        gather_issue(0, 0)
        pltpu.sync_copy(src_hbm.at[pl.ds(ebase + chunk, chunk)],
                        src_v.at[1])
        gather_issue(1, 1)
        issue_src(2, 2)
        issue_dl(0, 0)
        issue_dl(1, 1)

        def body_steps(i, b, wait_prev=True, pf_dl=True, pf_g=True,
                       pf_src=True):
            bp = (b + BUF - 1) % BUF
            if wait_prev:
                wait_scatter(bp)
            if pf_dl:
                issue_dl(i + 2, bp)
            if pf_g:
                wait_src(i + 2, bp)
                gather_issue(i + 2, bp)
            wait_gather(b)
            if pf_src:
                issue_src(i + 3, b)
            wait_dstlv(i, b)
            scale(rows[b], lv_v.at[b])
            scatter_issue(b)

        for i in range(H):
            body_steps(i, i % BUF, wait_prev=(i >= 1))

        def group(g, carry):
            i0 = H + BUF * g
            for r in range(BUF):
                body_steps(i0 + r, (H + r) % BUF)
            return carry

        lax.fori_loop(0, G, group, 0)

        t0 = nchunk - 3
        body_steps(t0, t0 % BUF, pf_src=False)
        body_steps(t0 + 1, (t0 + 1) % BUF, pf_dl=False, pf_g=False,
                   pf_src=False)
        body_steps(t0 + 2, (t0 + 2) % BUF, pf_dl=False, pf_g=False,
                   pf_src=False)
        wait_scatter((nchunk - 1) % BUF)
        plsc.subcore_barrier()
        pltpu.sync_copy(acc.at[pl.ds(sid * rps, rps)],
                        out_hbm.at[cid, pl.ds(sid * rps, rps)])
        if rem:
            @pl.when(sid == NS - 1)
            def _out_tail():
                pltpu.sync_copy(acc.at[pl.ds(rps * NS, rem)],
                                out_hbm.at[cid, pl.ds(rps * NS, rem)])

    return spmm


def _dense_layer(h, p, w_cat, b):
    n, d = h.shape
    rblk = 1000
    grid = (n // rblk,)

    def body(h_ref, p_ref, w_ref, b_ref, hn_ref, nrm_ref):
        lh = p_ref[0] + p_ref[1]
        hv = h_ref[...]
        cat = jnp.concatenate([lh + hv, hv * lh], axis=1)
        y = jnp.dot(cat, w_ref[...], preferred_element_type=jnp.float32)
        y = y + b_ref[...]
        y = jnp.where(y >= 0, y, 0.2 * y)
        hn_ref[...] = y
        ss = jnp.sum(y * y, axis=1, keepdims=True)
        nrm_ref[...] = y * lax.rsqrt(jnp.maximum(ss, 1e-12))

    row_spec = pl.BlockSpec((rblk, d), lambda i: (i, 0))
    return pl.pallas_call(
        body,
        grid=grid,
        in_specs=[
            row_spec,
            pl.BlockSpec((2, rblk, d), lambda i: (0, i, 0)),
            pl.BlockSpec((2 * d, d), lambda i: (0, 0)),
            pl.BlockSpec((1, d), lambda i: (0, 0)),
        ],
        out_specs=[row_spec, row_spec],
        out_shape=[
            jax.ShapeDtypeStruct((n, d), h.dtype),
            jax.ShapeDtypeStruct((n, d), h.dtype),
        ],
    )(h, p, w_cat, b)


def _dense_layer_final(h, p, w_cat, b, prevs):
    n, d = h.shape
    rblk = 1000
    grid = (n // rblk,)
    nprev = len(prevs)
    dout = (nprev + 1) * d

    def body(h_ref, p_ref, w_ref, b_ref, *rest):
        prev_refs, out_ref = rest[:nprev], rest[nprev]
        lh = p_ref[0] + p_ref[1]
        hv = h_ref[...]
        cat = jnp.concatenate([lh + hv, hv * lh], axis=1)
        y = jnp.dot(cat, w_ref[...], preferred_element_type=jnp.float32)
        y = y + b_ref[...]
        y = jnp.where(y >= 0, y, 0.2 * y)
        ss = jnp.sum(y * y, axis=1, keepdims=True)
        nrm = y * lax.rsqrt(jnp.maximum(ss, 1e-12))
        out_ref[...] = jnp.concatenate(
            [r[...] for r in prev_refs] + [nrm], axis=1)

    row_spec = pl.BlockSpec((rblk, d), lambda i: (i, 0))
    return pl.pallas_call(
        body,
        grid=grid,
        in_specs=[
            row_spec,
            pl.BlockSpec((2, rblk, d), lambda i: (0, i, 0)),
            pl.BlockSpec((2 * d, d), lambda i: (0, 0)),
            pl.BlockSpec((1, d), lambda i: (0, 0)),
        ] + [row_spec] * nprev,
        out_specs=pl.BlockSpec((rblk, dout), lambda i: (i, 0)),
        out_shape=jax.ShapeDtypeStruct((n, dout), h.dtype),
    )(h, p, w_cat, b, *prevs)


def kernel(x, edge_index, L_vals, W_gc, b_gc, W_bi, b_bi):
    n, d = x.shape
    e = L_vals.shape[0]
    k = W_gc.shape[0]
    src = edge_index[0]
    dst = edge_index[1]
    epw0 = e // NW
    pad = -(-epw0 // 128) * 128 - epw0
    if pad:
        src = jnp.pad(src.reshape(NW, epw0), ((0, 0), (0, pad))).reshape(-1)
        dst = jnp.pad(dst.reshape(NW, epw0), ((0, 0), (0, pad))).reshape(-1)
        L_vals_p = jnp.pad(L_vals.reshape(NW, epw0),
                           ((0, 0), (0, pad))).reshape(-1)
    else:
        L_vals_p = L_vals
    zeros = jnp.zeros(((n // NS) // 8 * 8, d), x.dtype)
    spmm = _make_spmm(n, d, e, x.dtype)

    h = x
    nrms = []
    for i in range(k):
        p = spmm(h, src, dst, L_vals_p, zeros)
        w_cat = jnp.concatenate([W_gc[i], W_bi[i]], axis=0)
        b = (b_gc[i] + b_bi[i]).reshape(1, d)
        if i < k - 1:
            h, nrm = _dense_layer(h, p, w_cat, b)
            nrms.append(nrm)
        else:
            out = _dense_layer_final(h, p, w_cat, b, [x] + nrms)
    return out


# speedup vs baseline: 1.3809x; 1.1484x over previous
import functools

import jax
import jax.numpy as jnp
from jax import lax
from jax.experimental import pallas as pl
from jax.experimental.pallas import tpu as pltpu
from jax.experimental.pallas import tpu_sc as plsc

NC = 2
NS = 16
NW = NC * NS
LANES = 16


def _make_spmm(n, d, e, dtype):
    chunk = 96
    epw = -(-(e // NW) // chunk) * chunk
    nchunk = epw // chunk
    rps = (n // NS) // 8 * 8
    rem = n - rps * NS
    mesh = plsc.VectorSubcoreMesh(core_axis_name="c", subcore_axis_name="s")

    BUF = 3
    H = next(h for h in range(BUF - 1, 3 * BUF)
             if (nchunk - 3 - h) % BUF == 0 and nchunk - 3 - h >= 0)
    G = (nchunk - 3 - H) // BUF

    @functools.partial(
        pl.kernel,
        mesh=mesh,
        out_type=jax.ShapeDtypeStruct((NC, n, d), dtype),
        scratch_types=[
            pltpu.VMEM_SHARED((n, d), dtype),
            pltpu.VMEM((BUF, chunk), jnp.int32),
            pltpu.VMEM((BUF, chunk), jnp.int32),
            pltpu.VMEM((BUF, chunk), dtype),
            pltpu.VMEM((chunk, d), dtype),
            pltpu.VMEM((chunk, d), dtype),
            pltpu.VMEM((chunk, d), dtype),
            pltpu.SemaphoreType.DMA((BUF,)),
            pltpu.SemaphoreType.DMA((BUF,)),
            pltpu.SemaphoreType.DMA((BUF,)),
            pltpu.SemaphoreType.DMA((BUF,)),
        ],
    )
    def spmm(h_hbm, src_hbm, dst_hbm, lv_hbm, z_hbm, out_hbm,
             acc, src_v, dst_v, lv_v, rows0, rows1, rows2,
             isem, jsem, gsem, ssem):
        cid = lax.axis_index("c")
        sid = lax.axis_index("s")
        wid = sid * NC + cid
        ebase = wid * epw
        rows = (rows0, rows1, rows2)

        pltpu.sync_copy(z_hbm.at[pl.ds(0, rps)], acc.at[pl.ds(sid * rps, rps)])
        if rem:
            @pl.when(sid == NS - 1)
            def _zero_tail():
                pltpu.sync_copy(z_hbm.at[pl.ds(0, rem)],
                                acc.at[pl.ds(rps * NS, rem)])
        plsc.subcore_barrier()

        def scale(rows_b, lv_ref):
            def grp(g, c2):
                w16 = lv_ref[pl.ds(g * LANES, LANES)]
                for jj in range(LANES):
                    wj = w16[jj]
                    row = g * LANES + jj
                    for j in range(d // LANES):
                        sl = pl.ds(j * LANES, LANES)
                        rows_b[row, sl] = rows_b[row, sl] * wj
                return c2
            lax.fori_loop(0, chunk // LANES, grp, 0)

        def gather_issue(i, b):
            pltpu.async_copy(h_hbm.at[src_v.at[b]], rows[b], gsem.at[b])

        def scatter_issue(b):
            pltpu.async_copy(rows[b], acc.at[dst_v.at[b]], ssem.at[b],
                             add=True)

        def wait_scatter(b):
            pltpu.make_async_copy(rows[b], acc.at[dst_v.at[b]],
                                  ssem.at[b]).wait()

        def wait_gather(b):
            pltpu.make_async_copy(h_hbm.at[src_v.at[b]], rows[b],
                                  gsem.at[b]).wait()

        def issue_src(i, b):
            off = ebase + i * chunk
            pltpu.async_copy(src_hbm.at[pl.ds(off, chunk)],
                             src_v.at[b], isem.at[b])

        def wait_src(i, b):
            off = ebase + i * chunk
            pltpu.make_async_copy(src_hbm.at[pl.ds(off, chunk)],
                                  src_v.at[b], isem.at[b]).wait()

        def issue_dl(i, b):
            off = ebase + i * chunk
            pltpu.async_copy(dst_hbm.at[pl.ds(off, chunk)],
                             dst_v.at[b], jsem.at[b])
            pltpu.async_copy(lv_hbm.at[pl.ds(off, chunk)],
                             lv_v.at[b], jsem.at[b])

        def wait_dstlv(i, b):
            off = ebase + i * chunk
            pltpu.make_async_copy(dst_hbm.at[pl.ds(off, chunk)],
                                  dst_v.at[b], jsem.at[b]).wait()
            pltpu.make_async_copy(lv_hbm.at[pl.ds(off, chunk)],
                                  lv_v.at[b], jsem.at[b]).wait()

        pltpu.sync_copy(src_hbm.at[pl.ds(ebase, chunk)], src_v.at[0])
        gather_issue(0, 0)
        pltpu.sync_copy(src_hbm.at[pl.ds(ebase + chunk, chunk)],
                        src_v.at[1])
        gather_issue(1, 1)
        issue_src(2, 2)
        issue_dl(0, 0)
        issue_dl(1, 1)

        def body_steps(i, b, wait_prev=True, pf_dl=True, pf_g=True,
                       pf_src=True):
            bp = (b + BUF - 1) % BUF
            if wait_prev:
                wait_scatter(bp)
            if pf_dl:
                issue_dl(i + 2, bp)
            if pf_g:
                wait_src(i + 2, bp)
                gather_issue(i + 2, bp)
            wait_gather(b)
            if pf_src:
                issue_src(i + 3, b)
            wait_dstlv(i, b)
            scale(rows[b], lv_v.at[b])
            scatter_issue(b)

        for i in range(H):
            body_steps(i, i % BUF, wait_prev=(i >= 1))

        def group(g, carry):
            i0 = H + BUF * g
            for r in range(BUF):
                body_steps(i0 + r, (H + r) % BUF)
            return carry

        lax.fori_loop(0, G, group, 0)

        t0 = nchunk - 3
        body_steps(t0, t0 % BUF, pf_src=False)
        body_steps(t0 + 1, (t0 + 1) % BUF, pf_dl=False, pf_g=False,
                   pf_src=False)
        body_steps(t0 + 2, (t0 + 2) % BUF, pf_dl=False, pf_g=False,
                   pf_src=False)
        wait_scatter((nchunk - 1) % BUF)
        plsc.subcore_barrier()
        pltpu.sync_copy(acc.at[pl.ds(sid * rps, rps)],
                        out_hbm.at[cid, pl.ds(sid * rps, rps)])
        if rem:
            @pl.when(sid == NS - 1)
            def _out_tail():
                pltpu.sync_copy(acc.at[pl.ds(rps * NS, rem)],
                                out_hbm.at[cid, pl.ds(rps * NS, rem)])

    return spmm


def _dense_layer(h, p, w_cat, b):
    n, d = h.shape
    rblk = 1000
    grid = (n // rblk,)

    def body(h_ref, p_ref, w_ref, b_ref, hn_ref, nrm_ref):
        lh = p_ref[0] + p_ref[1]
        hv = h_ref[...]
        cat = jnp.concatenate([lh + hv, hv * lh], axis=1)
        y = jnp.dot(cat, w_ref[...], preferred_element_type=jnp.float32)
        y = y + b_ref[...]
        y = jnp.where(y >= 0, y, 0.2 * y)
        hn_ref[...] = y
        ss = jnp.sum(y * y, axis=1, keepdims=True)
        nrm_ref[...] = y * lax.rsqrt(jnp.maximum(ss, 1e-12))

    row_spec = pl.BlockSpec((rblk, d), lambda i: (i, 0))
    return pl.pallas_call(
        body,
        grid=grid,
        in_specs=[
            row_spec,
            pl.BlockSpec((2, rblk, d), lambda i: (0, i, 0)),
            pl.BlockSpec((2 * d, d), lambda i: (0, 0)),
            pl.BlockSpec((1, d), lambda i: (0, 0)),
        ],
        out_specs=[row_spec, row_spec],
        out_shape=[
            jax.ShapeDtypeStruct((n, d), h.dtype),
            jax.ShapeDtypeStruct((n, d), h.dtype),
        ],
    )(h, p, w_cat, b)


def _dense_layer_final(h, p, w_cat, b, prevs):
    n, d = h.shape
    rblk = 1000
    grid = (n // rblk,)
    nprev = len(prevs)
    dout = (nprev + 1) * d

    def body(h_ref, p_ref, w_ref, b_ref, *rest):
        prev_refs, out_ref = rest[:nprev], rest[nprev]
        lh = p_ref[0] + p_ref[1]
        hv = h_ref[...]
        cat = jnp.concatenate([lh + hv, hv * lh], axis=1)
        y = jnp.dot(cat, w_ref[...], preferred_element_type=jnp.float32)
        y = y + b_ref[...]
        y = jnp.where(y >= 0, y, 0.2 * y)
        ss = jnp.sum(y * y, axis=1, keepdims=True)
        nrm = y * lax.rsqrt(jnp.maximum(ss, 1e-12))
        out_ref[...] = jnp.concatenate(
            [r[...] for r in prev_refs] + [nrm], axis=1)

    row_spec = pl.BlockSpec((rblk, d), lambda i: (i, 0))
    return pl.pallas_call(
        body,
        grid=grid,
        in_specs=[
            row_spec,
            pl.BlockSpec((2, rblk, d), lambda i: (0, i, 0)),
            pl.BlockSpec((2 * d, d), lambda i: (0, 0)),
            pl.BlockSpec((1, d), lambda i: (0, 0)),
        ] + [row_spec] * nprev,
        out_specs=pl.BlockSpec((rblk, dout), lambda i: (i, 0)),
        out_shape=jax.ShapeDtypeStruct((n, dout), h.dtype),
    )(h, p, w_cat, b, *prevs)


def kernel(x, edge_index, L_vals, W_gc, b_gc, W_bi, b_bi):
    n, d = x.shape
    e = L_vals.shape[0]
    k = W_gc.shape[0]
    src = edge_index[0]
    dst = edge_index[1]
    epw0 = e // NW
    pad = -(-epw0 // 96) * 96 - epw0
    if pad:
        src = jnp.pad(src.reshape(NW, epw0), ((0, 0), (0, pad))).reshape(-1)
        dst = jnp.pad(dst.reshape(NW, epw0), ((0, 0), (0, pad))).reshape(-1)
        L_vals_p = jnp.pad(L_vals.reshape(NW, epw0),
                           ((0, 0), (0, pad))).reshape(-1)
    else:
        L_vals_p = L_vals
    zeros = jnp.zeros(((n // NS) // 8 * 8, d), x.dtype)
    spmm = _make_spmm(n, d, e, x.dtype)

    h = x
    nrms = []
    for i in range(k):
        p = spmm(h, src, dst, L_vals_p, zeros)
        w_cat = jnp.concatenate([W_gc[i], W_bi[i]], axis=0)
        b = (b_gc[i] + b_bi[i]).reshape(1, d)
        if i < k - 1:
            h, nrm = _dense_layer(h, p, w_cat, b)
            nrms.append(nrm)
        else:
            out = _dense_layer_final(h, p, w_cat, b, [x] + nrms)
    return out
